# baseline (speedup 1.0000x reference)
import functools

import jax
import jax.numpy as jnp
from jax import lax
from jax.experimental import pallas as pl
from jax.experimental.pallas import tpu as pltpu
from jax.experimental.pallas import tpu_sc as plsc

H, I, E, T = 1024, 512, 64, 8
NTE = 8
NGEN = 2
NCAND = 16
NTOK = 4096
NA = 2 * NTOK
TILE = 256
NT = NA // TILE + (NTE + NGEN)
PAD = NT * TILE
BT = 512
GL = 128

_f32 = jnp.float32
_i32 = jnp.int32


def _candidates(tid, task_emb, task_router_w):
    tmask = lax.broadcasted_iota(_i32, (T, 1), 0) == tid
    tv = jnp.sum(jnp.where(tmask, task_emb, 0.0), axis=0, keepdims=True)
    ts = lax.dot_general(tv, task_router_w, (((1,), (1,)), ((), ())),
                         preferred_element_type=_f32)
    idx64 = lax.broadcasted_iota(_i32, (1, E), 1)
    lane16 = lax.broadcasted_iota(_i32, (1, NCAND), 1)
    sub16 = lax.broadcasted_iota(_i32, (NCAND, 1), 0)
    candv = jnp.full((1, NCAND), E - 1, _i32)
    candc = jnp.full((NCAND, 1), E - 1, _i32)
    s = ts
    for k in range(NTE):
        m = jnp.max(s, axis=1, keepdims=True)
        a = jnp.min(jnp.where(s == m, idx64, E), axis=1, keepdims=True)
        candv = jnp.where(lane16 == k, a, candv)
        candc = jnp.where(sub16 == k, a, candc)
        s = jnp.where(idx64 == a, -jnp.inf, s)
    for g in range(NGEN):
        val = E - NGEN + g
        candv = jnp.where(lane16 == NTE + g, val, candv)
        candc = jnp.where(sub16 == NTE + g, val, candc)
    eq = candc == candv
    rr = lax.broadcasted_iota(_i32, (NCAND, NCAND), 0)
    cc = lax.broadcasted_iota(_i32, (NCAND, NCAND), 1)
    dupv = jnp.sum(jnp.where(eq & (rr < cc), 1, 0),
                   axis=0, keepdims=True) > 0
    return tv, candv, candc, dupv


def _route_kernel(tid_ref, x_ref, temb_ref, trw_ref, gw_ref,
                  e1_ref, e2_ref, g1_ref, g2_ref):
    tid = tid_ref[0]
    tv, _, candc, _ = _candidates(tid, temb_ref[...], trw_ref[...])
    idx64 = lax.broadcasted_iota(_i32, (1, E), 1)
    act = jnp.sum(jnp.where(candc == idx64, 1, 0), axis=0, keepdims=True) > 0
    x = x_ref[...]
    lg = lax.dot_general(x + tv, gw_ref[...], (((1,), (1,)), ((), ())),
                         preferred_element_type=_f32)
    lg = jnp.where(act, lg, -jnp.inf)
    rid = lax.broadcasted_iota(_i32, (BT, E), 1)
    m1 = jnp.max(lg, axis=1, keepdims=True)
    a1 = jnp.min(jnp.where(lg == m1, rid, E), axis=1, keepdims=True)
    lg2 = jnp.where(rid == a1, -jnp.inf, lg)
    m2 = jnp.max(lg2, axis=1, keepdims=True)
    a2 = jnp.min(jnp.where(lg2 == m2, rid, E), axis=1, keepdims=True)
    z = jnp.sum(jnp.exp(lg - m1), axis=1, keepdims=True)
    p1 = 1.0 / z
    p2 = jnp.exp(m2 - m1) / z
    den = p1 + p2 + 1e-6
    e1_ref[...] = a1
    e2_ref[...] = a2
    lanesg = jnp.zeros((BT, GL), _f32)
    g1_ref[...] = lanesg + p1 / den
    g2_ref[...] = lanesg + p2 / den


def _perm_kernel(tid_ref, ef_ref, temb_ref, trw_ref, dest_ref, te_ref):
    tid = tid_ref[0]
    _, _, candc, dupv = _candidates(tid, temb_ref[...], trw_ref[...])
    sub16 = lax.broadcasted_iota(_i32, (NCAND, 1), 0)
    lane16 = lax.broadcasted_iota(_i32, (1, NCAND), 1)
    e = ef_ref[...]
    rr = lax.broadcasted_iota(_i32, (128, 128), 0)
    cc = lax.broadcasted_iota(_i32, (128, 128), 1)
    su128 = jnp.where(rr < cc, 1.0, 0.0)
    r64 = lax.broadcasted_iota(_i32, (64, 64), 0)
    c64 = lax.broadcasted_iota(_i32, (64, 64), 1)
    sl64 = jnp.where(c64 < r64, 1.0, 0.0)

    def cand_scalar(cidx):
        cs = jnp.sum(jnp.where(sub16 == cidx, candc, 0), axis=0, keepdims=True)
        nd = 1.0 - jnp.sum(jnp.where(lane16 == cidx,
                                     jnp.where(dupv, 1.0, 0.0), 0.0),
                           axis=1, keepdims=True)
        return cs, nd

    counts = jnp.zeros((NCAND, 1), _f32)
    for cidx in range(NCAND):
        cs, nd = cand_scalar(cidx)
        mask = jnp.where(e == cs, 1.0, 0.0) * nd
        cnt = jnp.sum(jnp.sum(mask, axis=1, keepdims=True), axis=0,
                      keepdims=True)
        counts = jnp.where(sub16 == cidx, cnt, counts)
    ci = counts.astype(_i32)
    ptiles = jnp.right_shift(ci + (TILE - 1), 8)
    padded = jnp.left_shift(ptiles, 8).astype(_f32)
    startf = lax.dot_general(
        jnp.where(c64[:NCAND, :NCAND] < r64[:NCAND, :NCAND], 1.0, 0.0),
        padded, (((1,), (0,)), ((), ())), preferred_element_type=_f32)

    dest = jnp.zeros((64, 128), _f32)
    te_acc = jnp.zeros((1, 128), _f32)
    li = lax.broadcasted_iota(_i32, (1, 128), 1).astype(_f32)
    ttot = jnp.sum(ptiles.astype(_f32), axis=0, keepdims=True)
    for cidx in range(NCAND):
        cs, nd = cand_scalar(cidx)
        mask = jnp.where(e == cs, 1.0, 0.0) * nd
        rowpre = lax.dot_general(mask, su128, (((1,), (0,)), ((), ())),
                                 preferred_element_type=_f32)
        rowtot = jnp.sum(mask, axis=1, keepdims=True)
        rowoff = lax.dot_general(sl64, rowtot, (((1,), (0,)), ((), ())),
                                 preferred_element_type=_f32)
        s_c = jnp.sum(jnp.where(sub16 == cidx, startf, 0.0),
                      axis=0, keepdims=True)
        dest = dest + mask * (s_c + rowpre + rowoff)
        st_t = s_c * (1.0 / TILE)
        en_t = st_t + jnp.sum(jnp.where(sub16 == cidx, ptiles.astype(_f32), 0.0),
                              axis=0, keepdims=True)
        cov = jnp.where((li >= st_t) & (li < en_t), 1.0, 0.0)
        te_acc = te_acc + cs.astype(_f32) * cov * nd
    te_acc = te_acc + float(E - 1) * jnp.where(li >= ttot, 1.0, 0.0)
    dest_ref[...] = dest.astype(_i32)
    te_ref[...] = (jnp.zeros((8, 128), _f32) + te_acc).astype(_i32)


def _ffn_kernel(te_ref, x_ref, gw_ref, uw_ref, dw_ref, gsc_ref, y_ref):
    x = x_ref[...].astype(_f32)
    hpre = lax.dot_general(x, gw_ref[0], (((1,), (1,)), ((), ())),
                           preferred_element_type=_f32)
    u = lax.dot_general(x, uw_ref[0], (((1,), (1,)), ((), ())),
                        preferred_element_type=_f32)
    a = hpre * jax.nn.sigmoid(hpre) * u
    a = a * gsc_ref[...][:, 0:1]
    y_ref[...] = lax.dot_general(a, dw_ref[0], (((1,), (1,)), ((), ())),
                                 preferred_element_type=_f32)


_NC = 2
_NS = 16


@functools.cache
def _sc_kernels():
    mesh = plsc.VectorSubcoreMesh(core_axis_name="c", subcore_axis_name="s")

    DCH = 32
    CCH = 16

    @functools.partial(
        pl.kernel, mesh=mesh,
        out_type=(jax.ShapeDtypeStruct((PAD, H // 2), _i32),
                  jax.ShapeDtypeStruct((PAD, GL), _f32)),
        scratch_types=[pltpu.VMEM((2, DCH, H // 2), _i32),
                       pltpu.VMEM((2, 2, DCH), _i32),
                       pltpu.VMEM((2, 2, DCH, GL), _f32),
                       pltpu.SemaphoreType.DMA,
                       pltpu.SemaphoreType.DMA,
                       pltpu.SemaphoreType.DMA,
                       pltpu.SemaphoreType.DMA],
    )
    def _dispatch(x_hbm, dest_hbm, g1_hbm, g2_hbm, xbuf_hbm, gsc_hbm,
                  xv, dv, gv, sl0, sl1, ss0, ss1):
        wid = lax.axis_index("s") * _NC + lax.axis_index("c")
        nch = 128 // DCH
        sls = (sl0, sl1)
        sss = (ss0, ss1)

        def load(c):
            b = c % 2
            base = wid * 128 + c * DCH
            hs = [pltpu.async_copy(x_hbm.at[pl.ds(base, DCH)], xv.at[b], sls[b])]
            for k, gh in enumerate((g1_hbm, g2_hbm)):
                hs.append(pltpu.async_copy(
                    dest_hbm.at[pl.ds(k * NTOK + base, DCH)], dv.at[b, k], sls[b]))
                hs.append(pltpu.async_copy(
                    gh.at[pl.ds(base, DCH)], gv.at[b, k], sls[b]))
            return hs

        pend_l = load(0)
        pend_s = [None, None]
        for c in range(nch):
            for hnd in pend_l:
                hnd.wait()
            b = c % 2
            if c + 1 < nch:
                b1 = (c + 1) % 2
                if pend_s[b1] is not None:
                    for hnd in pend_s[b1]:
                        hnd.wait()
                    pend_s[b1] = None
                pend_l = load(c + 1)
            if pend_s[b] is not None:
                for hnd in pend_s[b]:
                    hnd.wait()
                pend_s[b] = None
            hs = []
            for k in range(2):
                hs.append(pltpu.async_copy(xv.at[b], xbuf_hbm.at[dv.at[b, k]],
                                           sss[b]))
                hs.append(pltpu.async_copy(gv.at[b, k], gsc_hbm.at[dv.at[b, k]],
                                           sss[b]))
            pend_s[b] = hs
        for b in range(2):
            if pend_s[b] is not None:
                for hnd in pend_s[b]:
                    hnd.wait()

    @functools.partial(
        pl.kernel, mesh=mesh,
        out_type=jax.ShapeDtypeStruct((NTOK, H), _f32),
        scratch_types=[pltpu.VMEM((2, CCH, H), _f32),
                       pltpu.VMEM((2, CCH, H), _f32),
                       pltpu.VMEM((2, 2, CCH), _i32),
                       pltpu.SemaphoreType.DMA,
                       pltpu.SemaphoreType.DMA],
    )
    def _combine(y_hbm, dest_hbm, out_hbm, r0, r1, dv, sem0, sem1):
        wid = lax.axis_index("s") * _NC + lax.axis_index("c")
        nch = 128 // CCH
        sems = (sem0, sem1)

        def load(c):
            b = c % 2
            base = wid * 128 + c * CCH
            pltpu.sync_copy(dest_hbm.at[pl.ds(base, CCH)], dv.at[b, 0])
            pltpu.sync_copy(dest_hbm.at[pl.ds(NTOK + base, CCH)], dv.at[b, 1])
            return (pltpu.async_copy(y_hbm.at[dv.at[b, 0]], r0.at[b], sems[b]),
                    pltpu.async_copy(y_hbm.at[dv.at[b, 1]], r1.at[b], sems[b]))

        pend = load(0)
        for c in range(nch):
            for hnd in pend:
                hnd.wait()
            b = c % 2
            r0b = r0.at[b]
            r1b = r1.at[b]
            if c + 1 < nch:
                pend = load(c + 1)

            def body(i, carry):
                for q in range(H // 16):
                    r0b[i, pl.ds(q * 16, 16)] = (r0b[i, pl.ds(q * 16, 16)]
                                                 + r1b[i, pl.ds(q * 16, 16)])
                return carry

            lax.fori_loop(0, CCH, body, 0)
            base = wid * 128 + c * CCH
            pltpu.sync_copy(r0.at[b], out_hbm.at[pl.ds(base, CCH)])

    return _dispatch, _combine


def _route_call(tid, x, task_emb, task_router_w, gate_weight):
    spec = pltpu.PrefetchScalarGridSpec(
        num_scalar_prefetch=1,
        grid=(NTOK // BT,),
        in_specs=[
            pl.BlockSpec((BT, H), lambda i, tid_ref: (i, 0)),
            pl.BlockSpec((T, H), lambda i, tid_ref: (0, 0)),
            pl.BlockSpec((E, H), lambda i, tid_ref: (0, 0)),
            pl.BlockSpec((E, H), lambda i, tid_ref: (0, 0)),
        ],
        out_specs=[
            pl.BlockSpec((BT, 1), lambda i, tid_ref: (i, 0)),
            pl.BlockSpec((BT, 1), lambda i, tid_ref: (i, 0)),
            pl.BlockSpec((BT, GL), lambda i, tid_ref: (i, 0)),
            pl.BlockSpec((BT, GL), lambda i, tid_ref: (i, 0)),
        ],
    )
    return pl.pallas_call(
        _route_kernel, grid_spec=spec,
        out_shape=[jax.ShapeDtypeStruct((NTOK, 1), _i32),
                   jax.ShapeDtypeStruct((NTOK, 1), _i32),
                   jax.ShapeDtypeStruct((NTOK, GL), _f32),
                   jax.ShapeDtypeStruct((NTOK, GL), _f32)],
    )(tid, x, task_emb, task_router_w, gate_weight)


def _perm_call(tid, ef, task_emb, task_router_w):
    spec = pltpu.PrefetchScalarGridSpec(
        num_scalar_prefetch=1,
        grid=(1,),
        in_specs=[
            pl.BlockSpec((64, 128), lambda i, tid_ref: (0, 0)),
            pl.BlockSpec((T, H), lambda i, tid_ref: (0, 0)),
            pl.BlockSpec((E, H), lambda i, tid_ref: (0, 0)),
        ],
        out_specs=[
            pl.BlockSpec((64, 128), lambda i, tid_ref: (0, 0)),
            pl.BlockSpec((8, 128), lambda i, tid_ref: (0, 0)),
        ],
    )
    return pl.pallas_call(
        _perm_kernel, grid_spec=spec,
        out_shape=[jax.ShapeDtypeStruct((64, 128), _i32),
                   jax.ShapeDtypeStruct((8, 128), _i32)],
    )(tid, ef, task_emb, task_router_w)


def _ffn_call(te, xbuf, gate_w, up_w, down_w, gsc):
    spec = pltpu.PrefetchScalarGridSpec(
        num_scalar_prefetch=1,
        grid=(NT,),
        in_specs=[
            pl.BlockSpec((TILE, H), lambda i, te_ref: (i, 0)),
            pl.BlockSpec((1, I, H), lambda i, te_ref: (te_ref[i], 0, 0)),
            pl.BlockSpec((1, I, H), lambda i, te_ref: (te_ref[i], 0, 0)),
            pl.BlockSpec((1, H, I), lambda i, te_ref: (te_ref[i], 0, 0)),
            pl.BlockSpec((TILE, GL), lambda i, te_ref: (i, 0)),
        ],
        out_specs=pl.BlockSpec((TILE, H), lambda i, te_ref: (i, 0)),
    )
    return pl.pallas_call(
        _ffn_kernel, grid_spec=spec,
        out_shape=jax.ShapeDtypeStruct((PAD, H), _f32),
    )(te, xbuf, gate_w, up_w, down_w, gsc)


def kernel(hidden_states, task_id, task_router_w, gate_weight, task_emb,
           gate_w, up_w, down_w):
    bsz, seq_len, _ = hidden_states.shape
    x = hidden_states.reshape(bsz * seq_len, H)
    tid = jnp.asarray(task_id, _i32).reshape(1)
    e1, e2, g1, g2 = _route_call(tid, x, task_emb, task_router_w, gate_weight)
    ef = jnp.concatenate([e1, e2], axis=0).reshape(64, 128)
    dest64, te8 = _perm_call(tid, ef, task_emb, task_router_w)
    destf = dest64.reshape(NA)
    te = te8[0, :NT]
    dispatch_fn, combine_fn = _sc_kernels()
    xi = lax.bitcast_convert_type(
        x.astype(jnp.bfloat16).reshape(NTOK, H // 2, 2), _i32)
    xbuf_i, gsc = dispatch_fn(xi, destf, g1, g2)
    xbuf = lax.bitcast_convert_type(xbuf_i, jnp.bfloat16).reshape(PAD, H)
    y = _ffn_call(te, xbuf, gate_w, up_w, down_w, gsc)
    out = combine_fn(y, destf)
    return out.reshape(bsz, seq_len, H)


# speedup vs baseline: 2.5345x; 2.5345x over previous
import functools

import jax
import jax.numpy as jnp
from jax import lax
from jax.experimental import pallas as pl
from jax.experimental.pallas import tpu as pltpu
from jax.experimental.pallas import tpu_sc as plsc

H, I, E, T = 1024, 512, 64, 8
NTE = 8
NGEN = 2
NCAND = 16
NTOK = 4096
NA = 2 * NTOK
TILE = 256
NT = NA // TILE + (NTE + NGEN)
PAD = NT * TILE
BT = 512
GL = 128

_f32 = jnp.float32
_i32 = jnp.int32


def _candidates(tid, task_emb, task_router_w):
    tmask = lax.broadcasted_iota(_i32, (T, 1), 0) == tid
    tv = jnp.sum(jnp.where(tmask, task_emb, 0.0), axis=0, keepdims=True)
    ts = lax.dot_general(tv, task_router_w, (((1,), (1,)), ((), ())),
                         preferred_element_type=_f32)
    idx64 = lax.broadcasted_iota(_i32, (1, E), 1)
    lane16 = lax.broadcasted_iota(_i32, (1, NCAND), 1)
    sub16 = lax.broadcasted_iota(_i32, (NCAND, 1), 0)
    candv = jnp.full((1, NCAND), E - 1, _i32)
    candc = jnp.full((NCAND, 1), E - 1, _i32)
    s = ts
    for k in range(NTE):
        m = jnp.max(s, axis=1, keepdims=True)
        a = jnp.min(jnp.where(s == m, idx64, E), axis=1, keepdims=True)
        candv = jnp.where(lane16 == k, a, candv)
        candc = jnp.where(sub16 == k, a, candc)
        s = jnp.where(idx64 == a, -jnp.inf, s)
    for g in range(NGEN):
        val = E - NGEN + g
        candv = jnp.where(lane16 == NTE + g, val, candv)
        candc = jnp.where(sub16 == NTE + g, val, candc)
    eq = candc == candv
    rr = lax.broadcasted_iota(_i32, (NCAND, NCAND), 0)
    cc = lax.broadcasted_iota(_i32, (NCAND, NCAND), 1)
    dupv = jnp.sum(jnp.where(eq & (rr < cc), 1, 0),
                   axis=0, keepdims=True) > 0
    return tv, candv, candc, dupv


def _route_kernel(tid_ref, x_ref, temb_ref, trw_ref, gw_ref,
                  e1_ref, e2_ref, g1_ref, g2_ref):
    tid = tid_ref[0]
    tv, _, candc, _ = _candidates(tid, temb_ref[...], trw_ref[...])
    idx64 = lax.broadcasted_iota(_i32, (1, E), 1)
    act = jnp.sum(jnp.where(candc == idx64, 1, 0), axis=0, keepdims=True) > 0
    x = x_ref[...]
    lg = lax.dot_general(x + tv, gw_ref[...], (((1,), (1,)), ((), ())),
                         preferred_element_type=_f32)
    lg = jnp.where(act, lg, -jnp.inf)
    rid = lax.broadcasted_iota(_i32, (BT, E), 1)
    m1 = jnp.max(lg, axis=1, keepdims=True)
    a1 = jnp.min(jnp.where(lg == m1, rid, E), axis=1, keepdims=True)
    lg2 = jnp.where(rid == a1, -jnp.inf, lg)
    m2 = jnp.max(lg2, axis=1, keepdims=True)
    a2 = jnp.min(jnp.where(lg2 == m2, rid, E), axis=1, keepdims=True)
    z = jnp.sum(jnp.exp(lg - m1), axis=1, keepdims=True)
    p1 = 1.0 / z
    p2 = jnp.exp(m2 - m1) / z
    den = p1 + p2 + 1e-6
    e1_ref[...] = a1
    e2_ref[...] = a2
    lanesg = jnp.zeros((BT, GL), _f32)
    g1_ref[...] = lanesg + p1 / den
    g2_ref[...] = lanesg + p2 / den


def _perm_kernel(tid_ref, ef_ref, temb_ref, trw_ref, dest_ref, te_ref):
    tid = tid_ref[0]
    _, _, candc, dupv = _candidates(tid, temb_ref[...], trw_ref[...])
    sub16 = lax.broadcasted_iota(_i32, (NCAND, 1), 0)
    lane16 = lax.broadcasted_iota(_i32, (1, NCAND), 1)
    e = ef_ref[...]
    rr = lax.broadcasted_iota(_i32, (128, 128), 0)
    cc = lax.broadcasted_iota(_i32, (128, 128), 1)
    su128 = jnp.where(rr < cc, 1.0, 0.0)
    r64 = lax.broadcasted_iota(_i32, (64, 64), 0)
    c64 = lax.broadcasted_iota(_i32, (64, 64), 1)
    sl64 = jnp.where(c64 < r64, 1.0, 0.0)

    def cand_scalar(cidx):
        cs = jnp.sum(jnp.where(sub16 == cidx, candc, 0), axis=0, keepdims=True)
        nd = 1.0 - jnp.sum(jnp.where(lane16 == cidx,
                                     jnp.where(dupv, 1.0, 0.0), 0.0),
                           axis=1, keepdims=True)
        return cs, nd

    counts = jnp.zeros((NCAND, 1), _f32)
    for cidx in range(NCAND):
        cs, nd = cand_scalar(cidx)
        mask = jnp.where(e == cs, 1.0, 0.0) * nd
        cnt = jnp.sum(jnp.sum(mask, axis=1, keepdims=True), axis=0,
                      keepdims=True)
        counts = jnp.where(sub16 == cidx, cnt, counts)
    ci = counts.astype(_i32)
    ptiles = jnp.right_shift(ci + (TILE - 1), 8)
    padded = jnp.left_shift(ptiles, 8).astype(_f32)
    startf = lax.dot_general(
        jnp.where(c64[:NCAND, :NCAND] < r64[:NCAND, :NCAND], 1.0, 0.0),
        padded, (((1,), (0,)), ((), ())), preferred_element_type=_f32)

    dest = jnp.zeros((64, 128), _f32)
    te_acc = jnp.zeros((1, 128), _f32)
    li = lax.broadcasted_iota(_i32, (1, 128), 1).astype(_f32)
    ttot = jnp.sum(ptiles.astype(_f32), axis=0, keepdims=True)
    for cidx in range(NCAND):
        cs, nd = cand_scalar(cidx)
        mask = jnp.where(e == cs, 1.0, 0.0) * nd
        rowpre = lax.dot_general(mask, su128, (((1,), (0,)), ((), ())),
                                 preferred_element_type=_f32)
        rowtot = jnp.sum(mask, axis=1, keepdims=True)
        rowoff = lax.dot_general(sl64, rowtot, (((1,), (0,)), ((), ())),
                                 preferred_element_type=_f32)
        s_c = jnp.sum(jnp.where(sub16 == cidx, startf, 0.0),
                      axis=0, keepdims=True)
        dest = dest + mask * (s_c + rowpre + rowoff)
        st_t = s_c * (1.0 / TILE)
        en_t = st_t + jnp.sum(jnp.where(sub16 == cidx, ptiles.astype(_f32), 0.0),
                              axis=0, keepdims=True)
        cov = jnp.where((li >= st_t) & (li < en_t), 1.0, 0.0)
        te_acc = te_acc + cs.astype(_f32) * cov * nd
    te_acc = te_acc + float(E - 1) * jnp.where(li >= ttot, 1.0, 0.0)
    dest_ref[...] = dest.astype(_i32)
    te_ref[...] = (jnp.zeros((8, 128), _f32) + te_acc).astype(_i32)


def _ffn_kernel(te_ref, x_ref, gw_ref, uw_ref, dw_ref, gsc_ref, y_ref):
    x = x_ref[...]
    hpre = lax.dot_general(x, gw_ref[0], (((1,), (1,)), ((), ())),
                           preferred_element_type=_f32)
    u = lax.dot_general(x, uw_ref[0], (((1,), (1,)), ((), ())),
                        preferred_element_type=_f32)
    a = hpre * jax.nn.sigmoid(hpre) * u
    a = a * gsc_ref[...][:, 0:1]
    y_ref[...] = lax.dot_general(a, dw_ref[0], (((1,), (1,)), ((), ())),
                                 preferred_element_type=_f32)


_NC = 2
_NS = 16


@functools.cache
def _sc_kernels():
    mesh = plsc.VectorSubcoreMesh(core_axis_name="c", subcore_axis_name="s")

    DCH = 32
    CCH = 16

    @functools.partial(
        pl.kernel, mesh=mesh,
        out_type=(jax.ShapeDtypeStruct((PAD, H), _f32),
                  jax.ShapeDtypeStruct((PAD, GL), _f32)),
        scratch_types=[pltpu.VMEM((2, DCH, H), _f32),
                       pltpu.VMEM((2, 2, DCH), _i32),
                       pltpu.VMEM((2, 2, DCH, GL), _f32),
                       pltpu.SemaphoreType.DMA,
                       pltpu.SemaphoreType.DMA,
                       pltpu.SemaphoreType.DMA,
                       pltpu.SemaphoreType.DMA],
    )
    def _dispatch(x_hbm, dest_hbm, g1_hbm, g2_hbm, xbuf_hbm, gsc_hbm,
                  xv, dv, gv, sl0, sl1, ss0, ss1):
        wid = lax.axis_index("s") * _NC + lax.axis_index("c")
        nch = 128 // DCH
        sls = (sl0, sl1)
        sss = (ss0, ss1)

        def load(c):
            b = c % 2
            base = wid * 128 + c * DCH
            hs = [pltpu.async_copy(x_hbm.at[pl.ds(base, DCH)], xv.at[b], sls[b])]
            for k, gh in enumerate((g1_hbm, g2_hbm)):
                hs.append(pltpu.async_copy(
                    dest_hbm.at[pl.ds(k * NTOK + base, DCH)], dv.at[b, k], sls[b]))
                hs.append(pltpu.async_copy(
                    gh.at[pl.ds(base, DCH)], gv.at[b, k], sls[b]))
            return hs

        pend_l = load(0)
        pend_s = [None, None]
        for c in range(nch):
            for hnd in pend_l:
                hnd.wait()
            b = c % 2
            if c + 1 < nch:
                b1 = (c + 1) % 2
                if pend_s[b1] is not None:
                    for hnd in pend_s[b1]:
                        hnd.wait()
                    pend_s[b1] = None
                pend_l = load(c + 1)
            if pend_s[b] is not None:
                for hnd in pend_s[b]:
                    hnd.wait()
                pend_s[b] = None
            hs = []
            for k in range(2):
                hs.append(pltpu.async_copy(xv.at[b], xbuf_hbm.at[dv.at[b, k]],
                                           sss[b]))
                hs.append(pltpu.async_copy(gv.at[b, k], gsc_hbm.at[dv.at[b, k]],
                                           sss[b]))
            pend_s[b] = hs
        for b in range(2):
            if pend_s[b] is not None:
                for hnd in pend_s[b]:
                    hnd.wait()

    @functools.partial(
        pl.kernel, mesh=mesh,
        out_type=jax.ShapeDtypeStruct((NTOK, H), _f32),
        scratch_types=[pltpu.VMEM((2, CCH, H), _f32),
                       pltpu.VMEM((2, CCH, H), _f32),
                       pltpu.VMEM((2, 2, CCH), _i32),
                       pltpu.SemaphoreType.DMA,
                       pltpu.SemaphoreType.DMA],
    )
    def _combine(y_hbm, dest_hbm, out_hbm, r0, r1, dv, sem0, sem1):
        wid = lax.axis_index("s") * _NC + lax.axis_index("c")
        nch = 128 // CCH
        sems = (sem0, sem1)

        def load(c):
            b = c % 2
            base = wid * 128 + c * CCH
            pltpu.sync_copy(dest_hbm.at[pl.ds(base, CCH)], dv.at[b, 0])
            pltpu.sync_copy(dest_hbm.at[pl.ds(NTOK + base, CCH)], dv.at[b, 1])
            return (pltpu.async_copy(y_hbm.at[dv.at[b, 0]], r0.at[b], sems[b]),
                    pltpu.async_copy(y_hbm.at[dv.at[b, 1]], r1.at[b], sems[b]))

        pend = load(0)
        for c in range(nch):
            for hnd in pend:
                hnd.wait()
            b = c % 2
            r0b = r0.at[b]
            r1b = r1.at[b]
            if c + 1 < nch:
                pend = load(c + 1)

            def body(i, carry):
                for q in range(H // 16):
                    r0b[i, pl.ds(q * 16, 16)] = (r0b[i, pl.ds(q * 16, 16)]
                                                 + r1b[i, pl.ds(q * 16, 16)])
                return carry

            lax.fori_loop(0, CCH, body, 0)
            base = wid * 128 + c * CCH
            pltpu.sync_copy(r0.at[b], out_hbm.at[pl.ds(base, CCH)])

    return _dispatch, _combine


def _route_call(tid, x, task_emb, task_router_w, gate_weight):
    spec = pltpu.PrefetchScalarGridSpec(
        num_scalar_prefetch=1,
        grid=(NTOK // BT,),
        in_specs=[
            pl.BlockSpec((BT, H), lambda i, tid_ref: (i, 0)),
            pl.BlockSpec((T, H), lambda i, tid_ref: (0, 0)),
            pl.BlockSpec((E, H), lambda i, tid_ref: (0, 0)),
            pl.BlockSpec((E, H), lambda i, tid_ref: (0, 0)),
        ],
        out_specs=[
            pl.BlockSpec((BT, 1), lambda i, tid_ref: (i, 0)),
            pl.BlockSpec((BT, 1), lambda i, tid_ref: (i, 0)),
            pl.BlockSpec((BT, GL), lambda i, tid_ref: (i, 0)),
            pl.BlockSpec((BT, GL), lambda i, tid_ref: (i, 0)),
        ],
    )
    return pl.pallas_call(
        _route_kernel, grid_spec=spec,
        out_shape=[jax.ShapeDtypeStruct((NTOK, 1), _i32),
                   jax.ShapeDtypeStruct((NTOK, 1), _i32),
                   jax.ShapeDtypeStruct((NTOK, GL), _f32),
                   jax.ShapeDtypeStruct((NTOK, GL), _f32)],
    )(tid, x, task_emb, task_router_w, gate_weight)


def _perm_call(tid, ef, task_emb, task_router_w):
    spec = pltpu.PrefetchScalarGridSpec(
        num_scalar_prefetch=1,
        grid=(1,),
        in_specs=[
            pl.BlockSpec((64, 128), lambda i, tid_ref: (0, 0)),
            pl.BlockSpec((T, H), lambda i, tid_ref: (0, 0)),
            pl.BlockSpec((E, H), lambda i, tid_ref: (0, 0)),
        ],
        out_specs=[
            pl.BlockSpec((64, 128), lambda i, tid_ref: (0, 0)),
            pl.BlockSpec((8, 128), lambda i, tid_ref: (0, 0)),
        ],
    )
    return pl.pallas_call(
        _perm_kernel, grid_spec=spec,
        out_shape=[jax.ShapeDtypeStruct((64, 128), _i32),
                   jax.ShapeDtypeStruct((8, 128), _i32)],
    )(tid, ef, task_emb, task_router_w)


def _ffn_call(te, xbuf, gate_w, up_w, down_w, gsc):
    spec = pltpu.PrefetchScalarGridSpec(
        num_scalar_prefetch=1,
        grid=(NT,),
        in_specs=[
            pl.BlockSpec((TILE, H), lambda i, te_ref: (i, 0)),
            pl.BlockSpec((1, I, H), lambda i, te_ref: (te_ref[i], 0, 0)),
            pl.BlockSpec((1, I, H), lambda i, te_ref: (te_ref[i], 0, 0)),
            pl.BlockSpec((1, H, I), lambda i, te_ref: (te_ref[i], 0, 0)),
            pl.BlockSpec((TILE, GL), lambda i, te_ref: (i, 0)),
        ],
        out_specs=pl.BlockSpec((TILE, H), lambda i, te_ref: (i, 0)),
    )
    return pl.pallas_call(
        _ffn_kernel, grid_spec=spec,
        out_shape=jax.ShapeDtypeStruct((PAD, H), _f32),
    )(te, xbuf, gate_w, up_w, down_w, gsc)


def kernel(hidden_states, task_id, task_router_w, gate_weight, task_emb,
           gate_w, up_w, down_w):
    bsz, seq_len, _ = hidden_states.shape
    x = hidden_states.reshape(bsz * seq_len, H)
    tid = jnp.asarray(task_id, _i32).reshape(1)
    e1, e2, g1, g2 = _route_call(tid, x, task_emb, task_router_w, gate_weight)
    ef = jnp.concatenate([e1, e2], axis=0).reshape(64, 128)
    dest64, te8 = _perm_call(tid, ef, task_emb, task_router_w)
    destf = dest64.reshape(NA)
    te = te8[0, :NT]
    dispatch_fn, combine_fn = _sc_kernels()
    xbuf, gsc = dispatch_fn(x, destf, g1, g2)
    y = _ffn_call(te, xbuf, gate_w, up_w, down_w, gsc)
    out = combine_fn(y, destf)
    return out.reshape(bsz, seq_len, H)


# speedup vs baseline: 2.5538x; 1.0076x over previous
import functools

import jax
import jax.numpy as jnp
from jax import lax
from jax.experimental import pallas as pl
from jax.experimental.pallas import tpu as pltpu
from jax.experimental.pallas import tpu_sc as plsc

H, I, E, T = 1024, 512, 64, 8
NTE = 8
NGEN = 2
NCAND = 16
NTOK = 4096
NA = 2 * NTOK
TILE = 256
NT = NA // TILE + (NTE + NGEN)
PAD = NT * TILE
BT = 512
GL = 128

_f32 = jnp.float32
_i32 = jnp.int32


def _candidates(tid, task_emb, task_router_w):
    tmask = lax.broadcasted_iota(_i32, (T, 1), 0) == tid
    tv = jnp.sum(jnp.where(tmask, task_emb, 0.0), axis=0, keepdims=True)
    ts = lax.dot_general(tv, task_router_w, (((1,), (1,)), ((), ())),
                         preferred_element_type=_f32)
    idx64 = lax.broadcasted_iota(_i32, (1, E), 1)
    lane16 = lax.broadcasted_iota(_i32, (1, NCAND), 1)
    sub16 = lax.broadcasted_iota(_i32, (NCAND, 1), 0)
    candv = jnp.full((1, NCAND), E - 1, _i32)
    candc = jnp.full((NCAND, 1), E - 1, _i32)
    s = ts
    for k in range(NTE):
        m = jnp.max(s, axis=1, keepdims=True)
        a = jnp.min(jnp.where(s == m, idx64, E), axis=1, keepdims=True)
        candv = jnp.where(lane16 == k, a, candv)
        candc = jnp.where(sub16 == k, a, candc)
        s = jnp.where(idx64 == a, -jnp.inf, s)
    for g in range(NGEN):
        val = E - NGEN + g
        candv = jnp.where(lane16 == NTE + g, val, candv)
        candc = jnp.where(sub16 == NTE + g, val, candc)
    eq = candc == candv
    rr = lax.broadcasted_iota(_i32, (NCAND, NCAND), 0)
    cc = lax.broadcasted_iota(_i32, (NCAND, NCAND), 1)
    dupv = jnp.sum(jnp.where(eq & (rr < cc), 1, 0),
                   axis=0, keepdims=True) > 0
    return tv, candv, candc, dupv


def _route_kernel(tid_ref, x_ref, temb_ref, trw_ref, gw_ref,
                  e1_ref, e2_ref, g1_ref, g2_ref):
    tid = tid_ref[0]
    tv, _, candc, _ = _candidates(tid, temb_ref[...], trw_ref[...])
    idx64 = lax.broadcasted_iota(_i32, (1, E), 1)
    act = jnp.sum(jnp.where(candc == idx64, 1, 0), axis=0, keepdims=True) > 0
    x = x_ref[...]
    lg = lax.dot_general(x + tv, gw_ref[...], (((1,), (1,)), ((), ())),
                         preferred_element_type=_f32)
    lg = jnp.where(act, lg, -jnp.inf)
    rid = lax.broadcasted_iota(_i32, (BT, E), 1)
    m1 = jnp.max(lg, axis=1, keepdims=True)
    a1 = jnp.min(jnp.where(lg == m1, rid, E), axis=1, keepdims=True)
    lg2 = jnp.where(rid == a1, -jnp.inf, lg)
    m2 = jnp.max(lg2, axis=1, keepdims=True)
    a2 = jnp.min(jnp.where(lg2 == m2, rid, E), axis=1, keepdims=True)
    z = jnp.sum(jnp.exp(lg - m1), axis=1, keepdims=True)
    p1 = 1.0 / z
    p2 = jnp.exp(m2 - m1) / z
    den = p1 + p2 + 1e-6
    e1_ref[...] = a1
    e2_ref[...] = a2
    lanesg = jnp.zeros((BT, GL), _f32)
    g1_ref[...] = lanesg + p1 / den
    g2_ref[...] = lanesg + p2 / den


def _perm_kernel(tid_ref, ef_ref, temb_ref, trw_ref, dest_ref, te_ref):
    tid = tid_ref[0]
    _, _, candc, dupv = _candidates(tid, temb_ref[...], trw_ref[...])
    sub16 = lax.broadcasted_iota(_i32, (NCAND, 1), 0)
    lane16 = lax.broadcasted_iota(_i32, (1, NCAND), 1)
    e = ef_ref[...]
    rr = lax.broadcasted_iota(_i32, (128, 128), 0)
    cc = lax.broadcasted_iota(_i32, (128, 128), 1)
    su128 = jnp.where(rr < cc, 1.0, 0.0)
    r64 = lax.broadcasted_iota(_i32, (64, 64), 0)
    c64 = lax.broadcasted_iota(_i32, (64, 64), 1)
    sl64 = jnp.where(c64 < r64, 1.0, 0.0)

    def cand_scalar(cidx):
        cs = jnp.sum(jnp.where(sub16 == cidx, candc, 0), axis=0, keepdims=True)
        nd = 1.0 - jnp.sum(jnp.where(lane16 == cidx,
                                     jnp.where(dupv, 1.0, 0.0), 0.0),
                           axis=1, keepdims=True)
        return cs, nd

    counts = jnp.zeros((NCAND, 1), _f32)
    for cidx in range(NCAND):
        cs, nd = cand_scalar(cidx)
        mask = jnp.where(e == cs, 1.0, 0.0) * nd
        cnt = jnp.sum(jnp.sum(mask, axis=1, keepdims=True), axis=0,
                      keepdims=True)
        counts = jnp.where(sub16 == cidx, cnt, counts)
    ci = counts.astype(_i32)
    ptiles = jnp.right_shift(ci + (TILE - 1), 8)
    padded = jnp.left_shift(ptiles, 8).astype(_f32)
    startf = lax.dot_general(
        jnp.where(c64[:NCAND, :NCAND] < r64[:NCAND, :NCAND], 1.0, 0.0),
        padded, (((1,), (0,)), ((), ())), preferred_element_type=_f32)

    dest = jnp.zeros((64, 128), _f32)
    te_acc = jnp.zeros((1, 128), _f32)
    li = lax.broadcasted_iota(_i32, (1, 128), 1).astype(_f32)
    ttot = jnp.sum(ptiles.astype(_f32), axis=0, keepdims=True)
    for cidx in range(NCAND):
        cs, nd = cand_scalar(cidx)
        mask = jnp.where(e == cs, 1.0, 0.0) * nd
        rowpre = lax.dot_general(mask, su128, (((1,), (0,)), ((), ())),
                                 preferred_element_type=_f32)
        rowtot = jnp.sum(mask, axis=1, keepdims=True)
        rowoff = lax.dot_general(sl64, rowtot, (((1,), (0,)), ((), ())),
                                 preferred_element_type=_f32)
        s_c = jnp.sum(jnp.where(sub16 == cidx, startf, 0.0),
                      axis=0, keepdims=True)
        dest = dest + mask * (s_c + rowpre + rowoff)
        st_t = s_c * (1.0 / TILE)
        en_t = st_t + jnp.sum(jnp.where(sub16 == cidx, ptiles.astype(_f32), 0.0),
                              axis=0, keepdims=True)
        cov = jnp.where((li >= st_t) & (li < en_t), 1.0, 0.0)
        te_acc = te_acc + cs.astype(_f32) * cov * nd
    te_acc = te_acc + float(E - 1) * jnp.where(li >= ttot, 1.0, 0.0)
    dest_ref[...] = dest.astype(_i32)
    sub8 = lax.broadcasted_iota(_i32, (8, 128), 0)
    te_ref[...] = jnp.where(sub8 == 1, ttot,
                            jnp.zeros((8, 128), _f32) + te_acc).astype(_i32)


def _ffn_kernel(te_ref, nt_ref, x_ref, gw_ref, uw_ref, dw_ref, gsc_ref, y_ref):
    @pl.when(pl.program_id(0) < nt_ref[0])
    def _():
        x = x_ref[...]
        hpre = lax.dot_general(x, gw_ref[0], (((1,), (1,)), ((), ())),
                               preferred_element_type=_f32)
        u = lax.dot_general(x, uw_ref[0], (((1,), (1,)), ((), ())),
                            preferred_element_type=_f32)
        a = hpre * jax.nn.sigmoid(hpre) * u
        a = a * gsc_ref[...][:, 0:1]
        y_ref[...] = lax.dot_general(a, dw_ref[0], (((1,), (1,)), ((), ())),
                                     preferred_element_type=_f32)


_NC = 2
_NS = 16


@functools.cache
def _sc_kernels():
    mesh = plsc.VectorSubcoreMesh(core_axis_name="c", subcore_axis_name="s")

    DCH = 32
    CCH = 16

    @functools.partial(
        pl.kernel, mesh=mesh,
        out_type=(jax.ShapeDtypeStruct((PAD, H), _f32),
                  jax.ShapeDtypeStruct((PAD, GL), _f32)),
        scratch_types=[pltpu.VMEM((2, DCH, H), _f32),
                       pltpu.VMEM((2, 2, DCH), _i32),
                       pltpu.VMEM((2, 2, DCH, GL), _f32),
                       pltpu.SemaphoreType.DMA,
                       pltpu.SemaphoreType.DMA,
                       pltpu.SemaphoreType.DMA,
                       pltpu.SemaphoreType.DMA],
    )
    def _dispatch(x_hbm, dest_hbm, g1_hbm, g2_hbm, xbuf_hbm, gsc_hbm,
                  xv, dv, gv, sl0, sl1, ss0, ss1):
        wid = lax.axis_index("s") * _NC + lax.axis_index("c")
        nch = 128 // DCH
        sls = (sl0, sl1)
        sss = (ss0, ss1)

        def load(c):
            b = c % 2
            base = wid * 128 + c * DCH
            hs = [pltpu.async_copy(x_hbm.at[pl.ds(base, DCH)], xv.at[b], sls[b])]
            for k, gh in enumerate((g1_hbm, g2_hbm)):
                hs.append(pltpu.async_copy(
                    dest_hbm.at[pl.ds(k * NTOK + base, DCH)], dv.at[b, k], sls[b]))
                hs.append(pltpu.async_copy(
                    gh.at[pl.ds(base, DCH)], gv.at[b, k], sls[b]))
            return hs

        pend_l = load(0)
        pend_s = [None, None]
        for c in range(nch):
            for hnd in pend_l:
                hnd.wait()
            b = c % 2
            if c + 1 < nch:
                b1 = (c + 1) % 2
                if pend_s[b1] is not None:
                    for hnd in pend_s[b1]:
                        hnd.wait()
                    pend_s[b1] = None
                pend_l = load(c + 1)
            if pend_s[b] is not None:
                for hnd in pend_s[b]:
                    hnd.wait()
                pend_s[b] = None
            hs = []
            for k in range(2):
                hs.append(pltpu.async_copy(xv.at[b], xbuf_hbm.at[dv.at[b, k]],
                                           sss[b]))
                hs.append(pltpu.async_copy(gv.at[b, k], gsc_hbm.at[dv.at[b, k]],
                                           sss[b]))
            pend_s[b] = hs
        for b in range(2):
            if pend_s[b] is not None:
                for hnd in pend_s[b]:
                    hnd.wait()

    @functools.partial(
        pl.kernel, mesh=mesh,
        out_type=jax.ShapeDtypeStruct((NTOK, H), _f32),
        scratch_types=[pltpu.VMEM((2, CCH, H), _f32),
                       pltpu.VMEM((2, CCH, H), _f32),
                       pltpu.VMEM((2, 2, CCH), _i32),
                       pltpu.SemaphoreType.DMA,
                       pltpu.SemaphoreType.DMA],
    )
    def _combine(y_hbm, dest_hbm, out_hbm, r0, r1, dv, sem0, sem1):
        wid = lax.axis_index("s") * _NC + lax.axis_index("c")
        nch = 128 // CCH
        sems = (sem0, sem1)

        def load(c):
            b = c % 2
            base = wid * 128 + c * CCH
            pltpu.sync_copy(dest_hbm.at[pl.ds(base, CCH)], dv.at[b, 0])
            pltpu.sync_copy(dest_hbm.at[pl.ds(NTOK + base, CCH)], dv.at[b, 1])
            return (pltpu.async_copy(y_hbm.at[dv.at[b, 0]], r0.at[b], sems[b]),
                    pltpu.async_copy(y_hbm.at[dv.at[b, 1]], r1.at[b], sems[b]))

        pend = load(0)
        for c in range(nch):
            for hnd in pend:
                hnd.wait()
            b = c % 2
            r0b = r0.at[b]
            r1b = r1.at[b]
            if c + 1 < nch:
                pend = load(c + 1)

            def body(i, carry):
                for q in range(H // 16):
                    r0b[i, pl.ds(q * 16, 16)] = (r0b[i, pl.ds(q * 16, 16)]
                                                 + r1b[i, pl.ds(q * 16, 16)])
                return carry

            lax.fori_loop(0, CCH, body, 0)
            base = wid * 128 + c * CCH
            pltpu.sync_copy(r0.at[b], out_hbm.at[pl.ds(base, CCH)])

    return _dispatch, _combine


def _route_call(tid, x, task_emb, task_router_w, gate_weight):
    spec = pltpu.PrefetchScalarGridSpec(
        num_scalar_prefetch=1,
        grid=(NTOK // BT,),
        in_specs=[
            pl.BlockSpec((BT, H), lambda i, tid_ref: (i, 0)),
            pl.BlockSpec((T, H), lambda i, tid_ref: (0, 0)),
            pl.BlockSpec((E, H), lambda i, tid_ref: (0, 0)),
            pl.BlockSpec((E, H), lambda i, tid_ref: (0, 0)),
        ],
        out_specs=[
            pl.BlockSpec((BT, 1), lambda i, tid_ref: (i, 0)),
            pl.BlockSpec((BT, 1), lambda i, tid_ref: (i, 0)),
            pl.BlockSpec((BT, GL), lambda i, tid_ref: (i, 0)),
            pl.BlockSpec((BT, GL), lambda i, tid_ref: (i, 0)),
        ],
    )
    return pl.pallas_call(
        _route_kernel, grid_spec=spec,
        out_shape=[jax.ShapeDtypeStruct((NTOK, 1), _i32),
                   jax.ShapeDtypeStruct((NTOK, 1), _i32),
                   jax.ShapeDtypeStruct((NTOK, GL), _f32),
                   jax.ShapeDtypeStruct((NTOK, GL), _f32)],
    )(tid, x, task_emb, task_router_w, gate_weight)


def _perm_call(tid, ef, task_emb, task_router_w):
    spec = pltpu.PrefetchScalarGridSpec(
        num_scalar_prefetch=1,
        grid=(1,),
        in_specs=[
            pl.BlockSpec((64, 128), lambda i, tid_ref: (0, 0)),
            pl.BlockSpec((T, H), lambda i, tid_ref: (0, 0)),
            pl.BlockSpec((E, H), lambda i, tid_ref: (0, 0)),
        ],
        out_specs=[
            pl.BlockSpec((64, 128), lambda i, tid_ref: (0, 0)),
            pl.BlockSpec((8, 128), lambda i, tid_ref: (0, 0)),
        ],
    )
    return pl.pallas_call(
        _perm_kernel, grid_spec=spec,
        out_shape=[jax.ShapeDtypeStruct((64, 128), _i32),
                   jax.ShapeDtypeStruct((8, 128), _i32)],
    )(tid, ef, task_emb, task_router_w)


def _ffn_call(te, ntv, xbuf, gate_w, up_w, down_w, gsc):
    def clamp(i, nt_ref):
        return jnp.minimum(i, nt_ref[0] - 1)

    spec = pltpu.PrefetchScalarGridSpec(
        num_scalar_prefetch=2,
        grid=(NT,),
        in_specs=[
            pl.BlockSpec((TILE, H), lambda i, te, nt: (clamp(i, nt), 0)),
            pl.BlockSpec((1, I, H), lambda i, te, nt: (te[clamp(i, nt)], 0, 0)),
            pl.BlockSpec((1, I, H), lambda i, te, nt: (te[clamp(i, nt)], 0, 0)),
            pl.BlockSpec((1, H, I), lambda i, te, nt: (te[clamp(i, nt)], 0, 0)),
            pl.BlockSpec((TILE, GL), lambda i, te, nt: (clamp(i, nt), 0)),
        ],
        out_specs=pl.BlockSpec((TILE, H), lambda i, te, nt: (clamp(i, nt), 0)),
    )
    return pl.pallas_call(
        _ffn_kernel, grid_spec=spec,
        out_shape=jax.ShapeDtypeStruct((PAD, H), _f32),
    )(te, ntv, xbuf, gate_w, up_w, down_w, gsc)


def kernel(hidden_states, task_id, task_router_w, gate_weight, task_emb,
           gate_w, up_w, down_w):
    bsz, seq_len, _ = hidden_states.shape
    x = hidden_states.reshape(bsz * seq_len, H)
    tid = jnp.asarray(task_id, _i32).reshape(1)
    e1, e2, g1, g2 = _route_call(tid, x, task_emb, task_router_w, gate_weight)
    ef = jnp.concatenate([e1, e2], axis=0).reshape(64, 128)
    dest64, te8 = _perm_call(tid, ef, task_emb, task_router_w)
    destf = dest64.reshape(NA)
    te = te8[0, :NT]
    ntv = te8[1, :1]
    dispatch_fn, combine_fn = _sc_kernels()
    xbuf, gsc = dispatch_fn(x, destf, g1, g2)
    y = _ffn_call(te, ntv, xbuf, gate_w, up_w, down_w, gsc)
    out = combine_fn(y, destf)
    return out.reshape(bsz, seq_len, H)


# speedup vs baseline: 2.6760x; 1.0478x over previous
import functools

import jax
import jax.numpy as jnp
from jax import lax
from jax.experimental import pallas as pl
from jax.experimental.pallas import tpu as pltpu
from jax.experimental.pallas import tpu_sc as plsc

H, I, E, T = 1024, 512, 64, 8
NTE = 8
NGEN = 2
NCAND = 16
NTOK = 4096
NA = 2 * NTOK
TILE = 256
NT = NA // TILE + (NTE + NGEN)
PAD = NT * TILE
BT = 1024
GL = 128

_f32 = jnp.float32
_i32 = jnp.int32


def _candidates(tid, task_emb, task_router_w):
    tmask = lax.broadcasted_iota(_i32, (T, 1), 0) == tid
    tv = jnp.sum(jnp.where(tmask, task_emb, 0.0), axis=0, keepdims=True)
    ts = lax.dot_general(tv, task_router_w, (((1,), (1,)), ((), ())),
                         preferred_element_type=_f32)
    idx64 = lax.broadcasted_iota(_i32, (1, E), 1)
    lane16 = lax.broadcasted_iota(_i32, (1, NCAND), 1)
    sub16 = lax.broadcasted_iota(_i32, (NCAND, 1), 0)
    candv = jnp.full((1, NCAND), E - 1, _i32)
    candc = jnp.full((NCAND, 1), E - 1, _i32)
    s = ts
    for k in range(NTE):
        m = jnp.max(s, axis=1, keepdims=True)
        a = jnp.min(jnp.where(s == m, idx64, E), axis=1, keepdims=True)
        candv = jnp.where(lane16 == k, a, candv)
        candc = jnp.where(sub16 == k, a, candc)
        s = jnp.where(idx64 == a, -jnp.inf, s)
    for g in range(NGEN):
        val = E - NGEN + g
        candv = jnp.where(lane16 == NTE + g, val, candv)
        candc = jnp.where(sub16 == NTE + g, val, candc)
    eq = candc == candv
    rr = lax.broadcasted_iota(_i32, (NCAND, NCAND), 0)
    cc = lax.broadcasted_iota(_i32, (NCAND, NCAND), 1)
    dupv = jnp.sum(jnp.where(eq & (rr < cc), 1, 0),
                   axis=0, keepdims=True) > 0
    return tv, candv, candc, dupv


def _route_kernel(tid_ref, x_ref, temb_ref, trw_ref, gw_ref,
                  e1_ref, e2_ref, g1_ref, g2_ref):
    tid = tid_ref[0]
    tv, _, candc, _ = _candidates(tid, temb_ref[...], trw_ref[...])
    idx64 = lax.broadcasted_iota(_i32, (1, E), 1)
    act = jnp.sum(jnp.where(candc == idx64, 1, 0), axis=0, keepdims=True) > 0
    x = x_ref[...]
    lg = lax.dot_general(x + tv, gw_ref[...], (((1,), (1,)), ((), ())),
                         preferred_element_type=_f32)
    lg = jnp.where(act, lg, -jnp.inf)
    rid = lax.broadcasted_iota(_i32, (BT, E), 1)
    m1 = jnp.max(lg, axis=1, keepdims=True)
    a1 = jnp.min(jnp.where(lg == m1, rid, E), axis=1, keepdims=True)
    lg2 = jnp.where(rid == a1, -jnp.inf, lg)
    m2 = jnp.max(lg2, axis=1, keepdims=True)
    a2 = jnp.min(jnp.where(lg2 == m2, rid, E), axis=1, keepdims=True)
    z = jnp.sum(jnp.exp(lg - m1), axis=1, keepdims=True)
    p1 = 1.0 / z
    p2 = jnp.exp(m2 - m1) / z
    den = p1 + p2 + 1e-6
    e1_ref[...] = a1
    e2_ref[...] = a2
    lanesg = jnp.zeros((BT, GL), _f32)
    g1_ref[...] = lanesg + p1 / den
    g2_ref[...] = lanesg + p2 / den


def _perm_kernel(tid_ref, ef_ref, temb_ref, trw_ref, dest_ref, te_ref):
    tid = tid_ref[0]
    _, _, candc, dupv = _candidates(tid, temb_ref[...], trw_ref[...])
    sub16 = lax.broadcasted_iota(_i32, (NCAND, 1), 0)
    lane16 = lax.broadcasted_iota(_i32, (1, NCAND), 1)
    e = ef_ref[...]
    rr = lax.broadcasted_iota(_i32, (128, 128), 0)
    cc = lax.broadcasted_iota(_i32, (128, 128), 1)
    su128 = jnp.where(rr < cc, 1.0, 0.0)
    r64 = lax.broadcasted_iota(_i32, (64, 64), 0)
    c64 = lax.broadcasted_iota(_i32, (64, 64), 1)
    sl64 = jnp.where(c64 < r64, 1.0, 0.0)

    def cand_scalar(cidx):
        cs = jnp.sum(jnp.where(sub16 == cidx, candc, 0), axis=0, keepdims=True)
        nd = 1.0 - jnp.sum(jnp.where(lane16 == cidx,
                                     jnp.where(dupv, 1.0, 0.0), 0.0),
                           axis=1, keepdims=True)
        return cs, nd

    counts = jnp.zeros((NCAND, 1), _f32)
    for cidx in range(NCAND):
        cs, nd = cand_scalar(cidx)
        mask = jnp.where(e == cs, 1.0, 0.0) * nd
        cnt = jnp.sum(jnp.sum(mask, axis=1, keepdims=True), axis=0,
                      keepdims=True)
        counts = jnp.where(sub16 == cidx, cnt, counts)
    ci = counts.astype(_i32)
    ptiles = jnp.right_shift(ci + (TILE - 1), 8)
    padded = jnp.left_shift(ptiles, 8).astype(_f32)
    startf = lax.dot_general(
        jnp.where(c64[:NCAND, :NCAND] < r64[:NCAND, :NCAND], 1.0, 0.0),
        padded, (((1,), (0,)), ((), ())), preferred_element_type=_f32)

    dest = jnp.zeros((64, 128), _f32)
    te_acc = jnp.zeros((1, 128), _f32)
    li = lax.broadcasted_iota(_i32, (1, 128), 1).astype(_f32)
    ttot = jnp.sum(ptiles.astype(_f32), axis=0, keepdims=True)
    for cidx in range(NCAND):
        cs, nd = cand_scalar(cidx)
        mask = jnp.where(e == cs, 1.0, 0.0) * nd
        rowpre = lax.dot_general(mask, su128, (((1,), (0,)), ((), ())),
                                 preferred_element_type=_f32)
        rowtot = jnp.sum(mask, axis=1, keepdims=True)
        rowoff = lax.dot_general(sl64, rowtot, (((1,), (0,)), ((), ())),
                                 preferred_element_type=_f32)
        s_c = jnp.sum(jnp.where(sub16 == cidx, startf, 0.0),
                      axis=0, keepdims=True)
        dest = dest + mask * (s_c + rowpre + rowoff)
        st_t = s_c * (1.0 / TILE)
        en_t = st_t + jnp.sum(jnp.where(sub16 == cidx, ptiles.astype(_f32), 0.0),
                              axis=0, keepdims=True)
        cov = jnp.where((li >= st_t) & (li < en_t), 1.0, 0.0)
        te_acc = te_acc + cs.astype(_f32) * cov * nd
    te_acc = te_acc + float(E - 1) * jnp.where(li >= ttot, 1.0, 0.0)
    dest_ref[...] = dest.astype(_i32)
    sub8 = lax.broadcasted_iota(_i32, (8, 128), 0)
    te_ref[...] = jnp.where(sub8 == 1, ttot,
                            jnp.zeros((8, 128), _f32) + te_acc).astype(_i32)


def _ffn_kernel(te_ref, nt_ref, x_ref, gw_ref, uw_ref, dw_ref, gsc_ref, y_ref):
    @pl.when(pl.program_id(0) < nt_ref[0])
    def _():
        x = x_ref[...]
        hpre = lax.dot_general(x, gw_ref[0], (((1,), (1,)), ((), ())),
                               preferred_element_type=_f32)
        u = lax.dot_general(x, uw_ref[0], (((1,), (1,)), ((), ())),
                            preferred_element_type=_f32)
        a = hpre * jax.nn.sigmoid(hpre) * u
        a = a * gsc_ref[...][:, 0:1]
        y_ref[...] = lax.dot_general(a, dw_ref[0], (((1,), (1,)), ((), ())),
                                     preferred_element_type=_f32)


_NC = 2
_NS = 16


@functools.cache
def _sc_kernels():
    mesh = plsc.VectorSubcoreMesh(core_axis_name="c", subcore_axis_name="s")

    DCH = 32
    CCH = 16

    @functools.partial(
        pl.kernel, mesh=mesh,
        out_type=(jax.ShapeDtypeStruct((PAD, H), _f32),
                  jax.ShapeDtypeStruct((PAD, GL), _f32)),
        scratch_types=[pltpu.VMEM((2, DCH, H), _f32),
                       pltpu.VMEM((2, 2, DCH), _i32),
                       pltpu.VMEM((2, 2, DCH, GL), _f32),
                       pltpu.SemaphoreType.DMA,
                       pltpu.SemaphoreType.DMA,
                       pltpu.SemaphoreType.DMA,
                       pltpu.SemaphoreType.DMA],
    )
    def _dispatch(x_hbm, dest_hbm, g1_hbm, g2_hbm, xbuf_hbm, gsc_hbm,
                  xv, dv, gv, sl0, sl1, ss0, ss1):
        wid = lax.axis_index("s") * _NC + lax.axis_index("c")
        nch = 128 // DCH
        sls = (sl0, sl1)
        sss = (ss0, ss1)

        def load(c):
            b = c % 2
            base = wid * 128 + c * DCH
            hs = [pltpu.async_copy(x_hbm.at[pl.ds(base, DCH)], xv.at[b], sls[b])]
            for k, gh in enumerate((g1_hbm, g2_hbm)):
                hs.append(pltpu.async_copy(
                    dest_hbm.at[pl.ds(k * NTOK + base, DCH)], dv.at[b, k], sls[b]))
                hs.append(pltpu.async_copy(
                    gh.at[pl.ds(base, DCH)], gv.at[b, k], sls[b]))
            return hs

        pend_l = load(0)
        pend_s = [None, None]
        for c in range(nch):
            for hnd in pend_l:
                hnd.wait()
            b = c % 2
            if c + 1 < nch:
                b1 = (c + 1) % 2
                if pend_s[b1] is not None:
                    for hnd in pend_s[b1]:
                        hnd.wait()
                    pend_s[b1] = None
                pend_l = load(c + 1)
            if pend_s[b] is not None:
                for hnd in pend_s[b]:
                    hnd.wait()
                pend_s[b] = None
            hs = []
            for k in range(2):
                hs.append(pltpu.async_copy(xv.at[b], xbuf_hbm.at[dv.at[b, k]],
                                           sss[b]))
                hs.append(pltpu.async_copy(gv.at[b, k], gsc_hbm.at[dv.at[b, k]],
                                           sss[b]))
            pend_s[b] = hs
        for b in range(2):
            if pend_s[b] is not None:
                for hnd in pend_s[b]:
                    hnd.wait()

    @functools.partial(
        pl.kernel, mesh=mesh,
        out_type=jax.ShapeDtypeStruct((NTOK, H), _f32),
        scratch_types=[pltpu.VMEM((2, CCH, H), _f32),
                       pltpu.VMEM((2, CCH, H), _f32),
                       pltpu.VMEM((2, 2, CCH), _i32),
                       pltpu.SemaphoreType.DMA,
                       pltpu.SemaphoreType.DMA],
    )
    def _combine(y_hbm, dest_hbm, out_hbm, r0, r1, dv, sem0, sem1):
        wid = lax.axis_index("s") * _NC + lax.axis_index("c")
        nch = 128 // CCH
        sems = (sem0, sem1)

        def load(c):
            b = c % 2
            base = wid * 128 + c * CCH
            pltpu.sync_copy(dest_hbm.at[pl.ds(base, CCH)], dv.at[b, 0])
            pltpu.sync_copy(dest_hbm.at[pl.ds(NTOK + base, CCH)], dv.at[b, 1])
            return (pltpu.async_copy(y_hbm.at[dv.at[b, 0]], r0.at[b], sems[b]),
                    pltpu.async_copy(y_hbm.at[dv.at[b, 1]], r1.at[b], sems[b]))

        pend = load(0)
        for c in range(nch):
            for hnd in pend:
                hnd.wait()
            b = c % 2
            r0b = r0.at[b]
            r1b = r1.at[b]
            if c + 1 < nch:
                pend = load(c + 1)

            def body(i, carry):
                for q in range(H // 16):
                    r0b[i, pl.ds(q * 16, 16)] = (r0b[i, pl.ds(q * 16, 16)]
                                                 + r1b[i, pl.ds(q * 16, 16)])
                return carry

            lax.fori_loop(0, CCH, body, 0)
            base = wid * 128 + c * CCH
            pltpu.sync_copy(r0.at[b], out_hbm.at[pl.ds(base, CCH)])

    return _dispatch, _combine


def _route_call(tid, x, task_emb, task_router_w, gate_weight):
    spec = pltpu.PrefetchScalarGridSpec(
        num_scalar_prefetch=1,
        grid=(NTOK // BT,),
        in_specs=[
            pl.BlockSpec((BT, H), lambda i, tid_ref: (i, 0)),
            pl.BlockSpec((T, H), lambda i, tid_ref: (0, 0)),
            pl.BlockSpec((E, H), lambda i, tid_ref: (0, 0)),
            pl.BlockSpec((E, H), lambda i, tid_ref: (0, 0)),
        ],
        out_specs=[
            pl.BlockSpec((BT, 1), lambda i, tid_ref: (i, 0)),
            pl.BlockSpec((BT, 1), lambda i, tid_ref: (i, 0)),
            pl.BlockSpec((BT, GL), lambda i, tid_ref: (i, 0)),
            pl.BlockSpec((BT, GL), lambda i, tid_ref: (i, 0)),
        ],
    )
    return pl.pallas_call(
        _route_kernel, grid_spec=spec,
        out_shape=[jax.ShapeDtypeStruct((NTOK, 1), _i32),
                   jax.ShapeDtypeStruct((NTOK, 1), _i32),
                   jax.ShapeDtypeStruct((NTOK, GL), _f32),
                   jax.ShapeDtypeStruct((NTOK, GL), _f32)],
    )(tid, x, task_emb, task_router_w, gate_weight)


def _perm_call(tid, ef, task_emb, task_router_w):
    spec = pltpu.PrefetchScalarGridSpec(
        num_scalar_prefetch=1,
        grid=(1,),
        in_specs=[
            pl.BlockSpec((64, 128), lambda i, tid_ref: (0, 0)),
            pl.BlockSpec((T, H), lambda i, tid_ref: (0, 0)),
            pl.BlockSpec((E, H), lambda i, tid_ref: (0, 0)),
        ],
        out_specs=[
            pl.BlockSpec((64, 128), lambda i, tid_ref: (0, 0)),
            pl.BlockSpec((8, 128), lambda i, tid_ref: (0, 0)),
        ],
    )
    return pl.pallas_call(
        _perm_kernel, grid_spec=spec,
        out_shape=[jax.ShapeDtypeStruct((64, 128), _i32),
                   jax.ShapeDtypeStruct((8, 128), _i32)],
    )(tid, ef, task_emb, task_router_w)


def _ffn_call(te, ntv, xbuf, gate_w, up_w, down_w, gsc):
    def clamp(i, nt_ref):
        return jnp.minimum(i, nt_ref[0] - 1)

    spec = pltpu.PrefetchScalarGridSpec(
        num_scalar_prefetch=2,
        grid=(NT,),
        in_specs=[
            pl.BlockSpec((TILE, H), lambda i, te, nt: (clamp(i, nt), 0)),
            pl.BlockSpec((1, I, H), lambda i, te, nt: (te[clamp(i, nt)], 0, 0)),
            pl.BlockSpec((1, I, H), lambda i, te, nt: (te[clamp(i, nt)], 0, 0)),
            pl.BlockSpec((1, H, I), lambda i, te, nt: (te[clamp(i, nt)], 0, 0)),
            pl.BlockSpec((TILE, GL), lambda i, te, nt: (clamp(i, nt), 0)),
        ],
        out_specs=pl.BlockSpec((TILE, H), lambda i, te, nt: (clamp(i, nt), 0)),
    )
    return pl.pallas_call(
        _ffn_kernel, grid_spec=spec,
        out_shape=jax.ShapeDtypeStruct((PAD, H), _f32),
    )(te, ntv, xbuf, gate_w, up_w, down_w, gsc)


def kernel(hidden_states, task_id, task_router_w, gate_weight, task_emb,
           gate_w, up_w, down_w):
    bsz, seq_len, _ = hidden_states.shape
    x = hidden_states.reshape(bsz * seq_len, H)
    tid = jnp.asarray(task_id, _i32).reshape(1)
    e1, e2, g1, g2 = _route_call(tid, x, task_emb, task_router_w, gate_weight)
    ef = jnp.concatenate([e1, e2], axis=0).reshape(64, 128)
    dest64, te8 = _perm_call(tid, ef, task_emb, task_router_w)
    destf = dest64.reshape(NA)
    te = te8[0, :NT]
    ntv = te8[1, :1]
    dispatch_fn, combine_fn = _sc_kernels()
    xbuf, gsc = dispatch_fn(x, destf, g1, g2)
    y = _ffn_call(te, ntv, xbuf, gate_w, up_w, down_w, gsc)
    out = combine_fn(y, destf)
    return out.reshape(bsz, seq_len, H)
